# baseline (speedup 1.0000x reference)
import functools
import math

import jax
import jax.numpy as jnp
from jax.experimental import pallas as pl
from jax.experimental.pallas import tpu as pltpu

N = 100000
D = 128
SCALE = 100.0
C = SCALE * math.log2(math.e)


def _body(x_ref, k_ref, v_ref, acc_ref, l_ref, m_ref, *, nb):
    j = pl.program_id(1)

    @pl.when(j == 0)
    def _init():
        m_ref[...] = jnp.full_like(m_ref, -jnp.inf)
        l_ref[...] = jnp.zeros_like(l_ref)
        acc_ref[...] = jnp.zeros_like(acc_ref)

    sp = jax.lax.dot_general(
        x_ref[...], k_ref[...], (((1,), (1,)), ((), ())),
        preferred_element_type=jnp.float32,
    )
    m_prev = m_ref[...][:, :1]
    m_cur = jnp.max(sp, axis=1, keepdims=True)
    m_new = jnp.maximum(m_prev, m_cur)
    alpha = jnp.exp2((m_prev - m_new) * C)
    p = jnp.exp2((sp - m_new) * C)
    l_ref[...] = l_ref[...] * alpha + jnp.broadcast_to(
        jnp.sum(p, axis=1, keepdims=True), l_ref.shape)
    pv = jax.lax.dot_general(
        p.astype(jnp.bfloat16), v_ref[...], (((1,), (0,)), ((), ())),
        preferred_element_type=jnp.float32,
    )
    acc_ref[...] = acc_ref[...] * alpha + pv
    m_ref[...] = jnp.broadcast_to(m_new, m_ref.shape)


@jax.jit
def kernel(x, keys, vals):
    bq = 1024
    bn = 2000
    nq = x.shape[0] // bq
    nb = N // bn

    xb = x.astype(jnp.bfloat16)
    kb = keys.astype(jnp.bfloat16)
    vb = vals.astype(jnp.bfloat16)

    acc, l = pl.pallas_call(
        functools.partial(_body, nb=nb),
        grid=(nq, nb),
        in_specs=[
            pl.BlockSpec((bq, D), lambda i, j: (i, 0)),
            pl.BlockSpec((bn, D), lambda i, j: (j, 0)),
            pl.BlockSpec((bn, D), lambda i, j: (j, 0)),
        ],
        out_specs=[
            pl.BlockSpec((bq, D), lambda i, j: (i, 0)),
            pl.BlockSpec((bq, D), lambda i, j: (i, 0)),
        ],
        out_shape=[
            jax.ShapeDtypeStruct((x.shape[0], D), jnp.float32),
            jax.ShapeDtypeStruct((x.shape[0], D), jnp.float32),
        ],
        compiler_params=pltpu.CompilerParams(
            dimension_semantics=("parallel", "arbitrary")),
        scratch_shapes=[
            pltpu.VMEM((bq, D), jnp.float32),
        ],
    )(xb, kb, vb)
    return acc / l


# speedup vs baseline: 1.5108x; 1.5108x over previous
import math

import jax
import jax.numpy as jnp
from jax.experimental import pallas as pl
from jax.experimental.pallas import tpu as pltpu

N = 100000
D = 128
SCALE = 100.0
C = SCALE * math.log2(math.e)


def _flash_body(x_ref, k_ref, v_ref, acc_ref, l_ref, m_ref):
    j = pl.program_id(1)

    @pl.when(j == 0)
    def _init():
        m_ref[...] = jnp.full_like(m_ref, -jnp.inf)
        l_ref[...] = jnp.zeros_like(l_ref)
        acc_ref[...] = jnp.zeros_like(acc_ref)

    s = jax.lax.dot_general(
        x_ref[...], k_ref[...], (((1,), (1,)), ((), ())),
        preferred_element_type=jnp.float32,
    )

    m_prev = m_ref[...][:, :1]
    m_cur = jnp.max(s, axis=1, keepdims=True)
    m_new = jnp.maximum(m_prev, m_cur)
    alpha = jnp.exp2((m_prev - m_new) * C)
    p = jnp.exp2((s - m_new) * C)
    l_ref[...] = l_ref[...] * alpha + jnp.broadcast_to(
        jnp.sum(p, axis=1, keepdims=True), l_ref.shape)
    pv = jax.lax.dot_general(
        p, v_ref[...], (((1,), (0,)), ((), ())),
        preferred_element_type=jnp.float32,
    )
    acc_ref[...] = acc_ref[...] * alpha + pv
    m_ref[...] = jnp.broadcast_to(m_new, m_ref.shape)


@jax.jit
def kernel(x, keys, vals):
    bq = 1024
    bn = 4000
    nq = x.shape[0] // bq
    nb = N // bn

    acc, l = pl.pallas_call(
        _flash_body,
        grid=(nq, nb),
        in_specs=[
            pl.BlockSpec((bq, D), lambda i, j: (i, 0)),
            pl.BlockSpec((bn, D), lambda i, j: (j, 0)),
            pl.BlockSpec((bn, D), lambda i, j: (j, 0)),
        ],
        out_specs=[
            pl.BlockSpec((bq, D), lambda i, j: (i, 0)),
            pl.BlockSpec((bq, D), lambda i, j: (i, 0)),
        ],
        out_shape=[
            jax.ShapeDtypeStruct((x.shape[0], D), jnp.float32),
            jax.ShapeDtypeStruct((x.shape[0], D), jnp.float32),
        ],
        compiler_params=pltpu.CompilerParams(
            dimension_semantics=("parallel", "arbitrary")),
        scratch_shapes=[
            pltpu.VMEM((bq, D), jnp.float32),
        ],
    )(x, keys, vals)
    return acc / l
